# baseline (speedup 1.0000x reference)
import jax
import jax.numpy as jnp
from jax.experimental import pallas as pl
from jax.experimental.pallas import tpu as pltpu

_NEG = -1e30
_NSLOT = 8
_ROWS = 256


def _body(cats_ref, labels_ref, x_hbm, z_hbm, out_ref, xbuf, zbuf, acc_ref,
          xsem, zsem):
    B = x_hbm.shape[0]
    nsteps = 2 * B

    def start(cs, slot):
        b = jax.lax.div(cs, 2)
        r0 = jax.lax.rem(cs, 2) * _ROWS
        pltpu.make_async_copy(x_hbm.at[b, pl.ds(r0, _ROWS)], xbuf.at[slot],
                              xsem.at[slot]).start()
        pltpu.make_async_copy(z_hbm.at[b, pl.ds(r0, _ROWS)], zbuf.at[slot],
                              zsem.at[slot]).start()

    for cs in range(_NSLOT):
        start(cs, cs)

    def step(cs, carry):
        loss_sum, cur = carry
        slot = jax.lax.rem(cs, _NSLOT)
        pltpu.make_async_copy(x_hbm.at[0, pl.ds(0, _ROWS)], xbuf.at[slot],
                              xsem.at[slot]).wait()
        pltpu.make_async_copy(z_hbm.at[0, pl.ds(0, _ROWS)], zbuf.at[slot],
                              zsem.at[slot]).wait()
        x = xbuf[slot]
        z = zbuf[slot]
        b = jax.lax.div(cs, 2)
        cat = cats_ref[b]
        part = jnp.max(jnp.where(z == cat, x, _NEG))
        combined = jnp.maximum(cur, part)

        @pl.when(cs + _NSLOT < nsteps)
        def _next():
            start(cs + _NSLOT, slot)

        is_last = jax.lax.rem(cs, 2) == 1
        valid = (cat > 0) & (combined > -9e29)
        r = jnp.where(valid, combined, 0.0)
        y = labels_ref[b]
        per = jnp.maximum(r, 0.0) - r * y + jnp.log1p(jnp.exp(-jnp.abs(r)))
        loss_sum = jnp.where(is_last, loss_sum + per, loss_sum)
        cur = jnp.where(is_last, jnp.float32(_NEG), combined)
        return (loss_sum, cur)

    loss_sum, _ = jax.lax.fori_loop(0, nsteps, step,
                                    (jnp.float32(0.0), jnp.float32(_NEG)))
    out_ref[0, 0] = loss_sum / B


def kernel(pixel_logits, zone_patches, cats, labels):
    B, _, H, W = pixel_logits.shape
    logits = pixel_logits.reshape(B, H, W)

    grid_spec = pltpu.PrefetchScalarGridSpec(
        num_scalar_prefetch=2,
        grid=(),
        in_specs=[
            pl.BlockSpec(memory_space=pl.ANY),
            pl.BlockSpec(memory_space=pl.ANY),
        ],
        out_specs=pl.BlockSpec(memory_space=pltpu.SMEM),
        scratch_shapes=[
            pltpu.VMEM((_NSLOT, _ROWS, W), jnp.float32),
            pltpu.VMEM((_NSLOT, _ROWS, W), jnp.int32),
            pltpu.SMEM((1,), jnp.float32),
            pltpu.SemaphoreType.DMA((_NSLOT,)),
            pltpu.SemaphoreType.DMA((_NSLOT,)),
        ],
    )
    loss = pl.pallas_call(
        _body,
        grid_spec=grid_spec,
        out_shape=jax.ShapeDtypeStruct((1, 1), jnp.float32),
    )(cats, labels, logits, zone_patches)

    return loss[0, 0]
